# speedup vs baseline: 3.1495x; 3.1495x over previous
import functools

import jax
import jax.numpy as jnp
from jax.experimental import pallas as pl
from jax.experimental.pallas import tpu as pltpu

N_TOKENS = 8192
D_MODEL = 768
HIDDEN = 256
NUM_EXPERTS = 16
TOP_K = 2

BLK = 1024


def _fused_body(x_ref, w1_ref, b1_ref, w2_ref, b2_ref, w3_ref, b3_ref,
                gates_ref, idx_ref):
    h = jnp.dot(x_ref[...], w1_ref[...], preferred_element_type=jnp.float32)
    h = jnp.maximum(h + b1_ref[...], 0.0)
    h = jnp.dot(h, w2_ref[...], preferred_element_type=jnp.float32)
    h = jnp.maximum(h + b2_ref[...], 0.0)
    logits = jnp.dot(h, w3_ref[...], preferred_element_type=jnp.float32)
    logits = logits + b3_ref[...]

    e = jax.lax.broadcasted_iota(jnp.int32, logits.shape, 1)
    m1 = jnp.max(logits, axis=1, keepdims=True)
    i1 = jnp.min(jnp.where(logits == m1, e, NUM_EXPERTS), axis=1,
                 keepdims=True)
    mask1 = e == i1
    masked = jnp.where(mask1, -jnp.inf, logits)
    m2 = jnp.max(masked, axis=1, keepdims=True)
    i2 = jnp.min(jnp.where(masked == m2, e, NUM_EXPERTS), axis=1,
                 keepdims=True)
    mask2 = e == i2

    t = jnp.exp(m2 - m1)
    denom = 1.0 + t
    g1 = 1.0 / denom
    g2 = t / denom
    gates_ref[...] = jnp.where(mask1, g1, jnp.where(mask2, g2, 0.0))
    idx_ref[...] = jnp.concatenate([i1, i2], axis=1)


def kernel(x, W1, b1, W2, b2, W3, b3):
    n = x.shape[0]
    grid = (n // BLK,)
    b1r = b1.reshape(1, -1)
    b2r = b2.reshape(1, -1)
    b3r = b3.reshape(1, -1)
    full = lambda i: (0, 0)
    gates, idx = pl.pallas_call(
        _fused_body,
        grid=grid,
        in_specs=[
            pl.BlockSpec((BLK, D_MODEL), lambda i: (i, 0)),
            pl.BlockSpec((D_MODEL, HIDDEN), full),
            pl.BlockSpec((1, HIDDEN), full),
            pl.BlockSpec((HIDDEN, HIDDEN // 2), full),
            pl.BlockSpec((1, HIDDEN // 2), full),
            pl.BlockSpec((HIDDEN // 2, NUM_EXPERTS), full),
            pl.BlockSpec((1, NUM_EXPERTS), full),
        ],
        out_specs=[
            pl.BlockSpec((BLK, NUM_EXPERTS), lambda i: (i, 0)),
            pl.BlockSpec((BLK, TOP_K), lambda i: (i, 0)),
        ],
        out_shape=[
            jax.ShapeDtypeStruct((n, NUM_EXPERTS), jnp.float32),
            jax.ShapeDtypeStruct((n, TOP_K), jnp.int32),
        ],
    )(x, W1, b1r, W2, b2r, W3, b3r)
    return gates, idx
